# baseline (speedup 1.0000x reference)
import jax
import jax.numpy as jnp
from jax.experimental import pallas as pl

_N = 50000
_G = 16
_C = 8
_NUM_FEAT = 123
_NUM_CFG_FEAT = 18
_NUM_OPS = 125
_DIM = 128

_BLOCK = 1000


def _fused_kernel(code_ref, bidx_ref, opf_ref, cfg_ref, opw_ref, cfgw_ref,
                  emb_ref, wop_ref, bop_ref, wcfg_ref, bcfg_ref, out_ref):
    code = code_ref[0, :, :]
    ohT_op = (code == jax.lax.broadcasted_iota(jnp.int32, (_NUM_OPS, 1), 0)
              ).astype(jnp.float32)
    row = jax.lax.dot_general(
        ohT_op, emb_ref[:, :], (((0,), (0,)), ((), ())),
        preferred_element_type=jnp.float32)
    sq = jnp.sum(row * row, axis=1, keepdims=True)
    scale = jnp.where(sq > 1.0, jax.lax.rsqrt(sq), 1.0)
    op_emb = opw_ref[0, 0] * (row * scale)

    x = (jax.lax.dot_general(
            opf_ref[:, :], wop_ref[:, :], (((1,), (1,)), ((), ())),
            preferred_element_type=jnp.float32)
         + bop_ref[0, :][None, :] + op_emb)

    ohT_g = (bidx_ref[0, :, :] ==
             jax.lax.broadcasted_iota(jnp.int32, (_G, 1), 0)
             ).astype(jnp.float32)
    scaled_cfg = cfg_ref[:, :, :] * cfgw_ref[0, :][None, None, :]
    cfg_flat = scaled_cfg.reshape(_G * _C, _NUM_CFG_FEAT)
    cfg_all = (jax.lax.dot_general(
                   cfg_flat, wcfg_ref[:, :], (((1,), (1,)), ((), ())),
                   preferred_element_type=jnp.float32)
               + bcfg_ref[0, :][None, :]).reshape(_G, _C, _DIM)
    cfg_pn_flat = jax.lax.dot_general(
        ohT_g, cfg_all.reshape(_G, _C * _DIM), (((0,), (0,)), ((), ())),
        preferred_element_type=jnp.float32)
    res_flat = cfg_pn_flat + jnp.concatenate([x] * _C, axis=1)
    out_ref[:, :, :] = res_flat.reshape(_BLOCK, _C, _DIM)


def kernel(op_code, op_feats, config_feats, batch_idx, op_weights,
           config_weights, emb_table, W_op, b_op, W_cfg, b_cfg):
    n = op_feats.shape[0]
    nb = n // _BLOCK
    code3 = op_code.reshape(nb, 1, _BLOCK).astype(jnp.int32)
    bidx3 = batch_idx.reshape(nb, 1, _BLOCK).astype(jnp.int32)
    cfgw2 = config_weights.reshape(1, _NUM_CFG_FEAT)
    bop2 = b_op.reshape(1, _DIM)
    bcfg2 = b_cfg.reshape(1, _DIM)

    grid = (nb,)

    def whole(shape):
        return pl.BlockSpec(shape, lambda i: (0,) * len(shape))

    out = pl.pallas_call(
        _fused_kernel,
        grid=grid,
        in_specs=[
            pl.BlockSpec((1, 1, _BLOCK), lambda i: (i, 0, 0)),
            pl.BlockSpec((1, 1, _BLOCK), lambda i: (i, 0, 0)),
            pl.BlockSpec((_BLOCK, _NUM_FEAT), lambda i: (i, 0)),
            whole((_G, _C, _NUM_CFG_FEAT)),
            whole((1, 1)),
            whole((1, _NUM_CFG_FEAT)),
            whole((_NUM_OPS, _DIM)),
            whole((_DIM, _NUM_FEAT)),
            whole((1, _DIM)),
            whole((_DIM, _NUM_CFG_FEAT)),
            whole((1, _DIM)),
        ],
        out_specs=pl.BlockSpec((_BLOCK, _C, _DIM), lambda i: (i, 0, 0)),
        out_shape=jax.ShapeDtypeStruct((n, _C, _DIM), jnp.float32),
    )(code3, bidx3, op_feats, config_feats, op_weights, cfgw2,
      emb_table, W_op, bop2, W_cfg, bcfg2)
    return out


# speedup vs baseline: 1.2045x; 1.2045x over previous
import jax
import jax.numpy as jnp
from jax.experimental import pallas as pl

_N = 50000
_G = 16
_C = 8
_NUM_FEAT = 123
_NUM_CFG_FEAT = 18
_NUM_OPS = 125
_DIM = 128

_BLOCK = 2000


def _fused_kernel(code_ref, bidx_ref, opf_ref, cfg_ref, opw_ref, cfgw_ref,
                  emb_ref, wop_ref, bop_ref, wcfg_ref, bcfg_ref, out_ref):
    code = code_ref[0, :, :]
    ohT_op = (code == jax.lax.broadcasted_iota(jnp.int32, (_NUM_OPS, 1), 0)
              ).astype(jnp.float32)
    row = jax.lax.dot_general(
        ohT_op, emb_ref[:, :], (((0,), (0,)), ((), ())),
        preferred_element_type=jnp.float32)
    sq = jnp.sum(row * row, axis=1, keepdims=True)
    scale = jnp.where(sq > 1.0, jax.lax.rsqrt(sq), 1.0)
    op_emb = opw_ref[0, 0] * (row * scale)

    x = (jax.lax.dot_general(
            opf_ref[:, :], wop_ref[:, :], (((1,), (1,)), ((), ())),
            preferred_element_type=jnp.float32)
         + bop_ref[0, :][None, :] + op_emb)

    ohT_g = (bidx_ref[0, :, :] ==
             jax.lax.broadcasted_iota(jnp.int32, (_G, 1), 0)
             ).astype(jnp.float32)
    scaled_cfg = cfg_ref[:, :, :] * cfgw_ref[0, :][None, None, :]
    cfg_flat = scaled_cfg.reshape(_G * _C, _NUM_CFG_FEAT)
    cfg_all = (jax.lax.dot_general(
                   cfg_flat, wcfg_ref[:, :], (((1,), (1,)), ((), ())),
                   preferred_element_type=jnp.float32)
               + bcfg_ref[0, :][None, :]).reshape(_G, _C, _DIM)
    cfg_pn_flat = jax.lax.dot_general(
        ohT_g, cfg_all.reshape(_G, _C * _DIM), (((0,), (0,)), ((), ())),
        preferred_element_type=jnp.float32)
    res_flat = cfg_pn_flat + jnp.concatenate([x] * _C, axis=1)
    out_ref[:, :, :] = res_flat.reshape(_BLOCK, _C, _DIM)


def kernel(op_code, op_feats, config_feats, batch_idx, op_weights,
           config_weights, emb_table, W_op, b_op, W_cfg, b_cfg):
    n = op_feats.shape[0]
    nb = n // _BLOCK
    code3 = op_code.reshape(nb, 1, _BLOCK).astype(jnp.int32)
    bidx3 = batch_idx.reshape(nb, 1, _BLOCK).astype(jnp.int32)
    cfgw2 = config_weights.reshape(1, _NUM_CFG_FEAT)
    bop2 = b_op.reshape(1, _DIM)
    bcfg2 = b_cfg.reshape(1, _DIM)

    grid = (nb,)

    def whole(shape):
        return pl.BlockSpec(shape, lambda i: (0,) * len(shape))

    out = pl.pallas_call(
        _fused_kernel,
        grid=grid,
        in_specs=[
            pl.BlockSpec((1, 1, _BLOCK), lambda i: (i, 0, 0)),
            pl.BlockSpec((1, 1, _BLOCK), lambda i: (i, 0, 0)),
            pl.BlockSpec((_BLOCK, _NUM_FEAT), lambda i: (i, 0)),
            whole((_G, _C, _NUM_CFG_FEAT)),
            whole((1, 1)),
            whole((1, _NUM_CFG_FEAT)),
            whole((_NUM_OPS, _DIM)),
            whole((_DIM, _NUM_FEAT)),
            whole((1, _DIM)),
            whole((_DIM, _NUM_CFG_FEAT)),
            whole((1, _DIM)),
        ],
        out_specs=pl.BlockSpec((_BLOCK, _C, _DIM), lambda i: (i, 0, 0)),
        out_shape=jax.ShapeDtypeStruct((n, _C, _DIM), jnp.float32),
    )(code3, bidx3, op_feats, config_feats, op_weights, cfgw2,
      emb_table, W_op, bop2, W_cfg, bcfg2)
    return out


# speedup vs baseline: 1.2045x; 1.0001x over previous
import jax
import jax.numpy as jnp
from jax.experimental import pallas as pl

_G = 16
_C = 8
_NUM_FEAT = 123
_NUM_CFG_FEAT = 18
_NUM_OPS = 125
_DIM = 128

_BLOCK = 2000


def _fused_kernel(code_ref, bidx_ref, opf_ref, cfg_ref, opw_ref, cfgw_ref,
                  emb_ref, wop_ref, bop_ref, wcfg_ref, bcfg_ref, out_ref):
    code = code_ref[0, :, :]
    ohT_op = (code == jax.lax.broadcasted_iota(jnp.int32, (_NUM_OPS, 1), 0)
              ).astype(jnp.float32)
    row = jax.lax.dot_general(
        ohT_op, emb_ref[:, :], (((0,), (0,)), ((), ())),
        preferred_element_type=jnp.float32)
    sq = jnp.sum(row * row, axis=1, keepdims=True)
    scale = jnp.where(sq > 1.0, jax.lax.rsqrt(sq), 1.0)
    op_emb = opw_ref[0, 0] * (row * scale)

    x = (jax.lax.dot_general(
            opf_ref[:, :], wop_ref[:, :], (((1,), (1,)), ((), ())),
            preferred_element_type=jnp.float32)
         + bop_ref[0, :][None, :] + op_emb)

    ohT_g = (bidx_ref[0, :, :] ==
             jax.lax.broadcasted_iota(jnp.int32, (_G, 1), 0)
             ).astype(jnp.float32)
    scaled_cfg = cfg_ref[:, :, :] * cfgw_ref[0, :][None, None, :]
    cfg_flat = scaled_cfg.reshape(_G * _C, _NUM_CFG_FEAT)
    cfg_all = (jax.lax.dot_general(
                   cfg_flat, wcfg_ref[:, :], (((1,), (1,)), ((), ())),
                   preferred_element_type=jnp.float32)
               + bcfg_ref[0, :][None, :]).reshape(_G, _C, _DIM)
    cfg_pn_flat = jax.lax.dot_general(
        ohT_g, cfg_all.reshape(_G, _C * _DIM), (((0,), (0,)), ((), ())),
        preferred_element_type=jnp.float32)
    res_flat = cfg_pn_flat + jnp.concatenate([x] * _C, axis=1)
    out_ref[:, :, :] = res_flat.reshape(_BLOCK, _C, _DIM)


def kernel(op_code, op_feats, config_feats, batch_idx, op_weights,
           config_weights, emb_table, W_op, b_op, W_cfg, b_cfg):
    n = op_feats.shape[0]
    nb = n // _BLOCK
    code3 = op_code.reshape(nb, 1, _BLOCK).astype(jnp.int32)
    bidx3 = batch_idx.reshape(nb, 1, _BLOCK).astype(jnp.int32)
    cfgw2 = config_weights.reshape(1, _NUM_CFG_FEAT)
    bop2 = b_op.reshape(1, _DIM)
    bcfg2 = b_cfg.reshape(1, _DIM)

    grid = (nb,)

    def whole(shape):
        return pl.BlockSpec(shape, lambda i: (0,) * len(shape))

    out = pl.pallas_call(
        _fused_kernel,
        grid=grid,
        in_specs=[
            pl.BlockSpec((1, 1, _BLOCK), lambda i: (i, 0, 0)),
            pl.BlockSpec((1, 1, _BLOCK), lambda i: (i, 0, 0)),
            pl.BlockSpec((_BLOCK, _NUM_FEAT), lambda i: (i, 0)),
            whole((_G, _C, _NUM_CFG_FEAT)),
            whole((1, 1)),
            whole((1, _NUM_CFG_FEAT)),
            whole((_NUM_OPS, _DIM)),
            whole((_DIM, _NUM_FEAT)),
            whole((1, _DIM)),
            whole((_DIM, _NUM_CFG_FEAT)),
            whole((1, _DIM)),
        ],
        out_specs=pl.BlockSpec((_BLOCK, _C, _DIM), lambda i: (i, 0, 0)),
        out_shape=jax.ShapeDtypeStruct((n, _C, _DIM), jnp.float32),
    )(code3, bidx3, op_feats, config_feats, op_weights, cfgw2,
      emb_table, W_op, bop2, W_cfg, bcfg2)
    return out


# speedup vs baseline: 1.2205x; 1.0132x over previous
import jax
import jax.numpy as jnp
from jax.experimental import pallas as pl

_G = 16
_C = 8
_NUM_FEAT = 123
_NUM_CFG_FEAT = 18
_NUM_OPS = 125
_DIM = 128

_BLOCK = 4000


def _fused_kernel(code_ref, bidx_ref, opf_ref, cfg_ref, opw_ref, cfgw_ref,
                  emb_ref, wop_ref, bop_ref, wcfg_ref, bcfg_ref, out_ref):
    code = code_ref[0, :, :]
    ohT_op = (code == jax.lax.broadcasted_iota(jnp.int32, (_NUM_OPS, 1), 0)
              ).astype(jnp.float32)
    row = jax.lax.dot_general(
        ohT_op, emb_ref[:, :], (((0,), (0,)), ((), ())),
        preferred_element_type=jnp.float32)
    sq = jnp.sum(row * row, axis=1, keepdims=True)
    scale = jnp.where(sq > 1.0, jax.lax.rsqrt(sq), 1.0)
    op_emb = opw_ref[0, 0] * (row * scale)

    x = (jax.lax.dot_general(
            opf_ref[:, :], wop_ref[:, :], (((1,), (1,)), ((), ())),
            preferred_element_type=jnp.float32)
         + bop_ref[0, :][None, :] + op_emb)

    ohT_g = (bidx_ref[0, :, :] ==
             jax.lax.broadcasted_iota(jnp.int32, (_G, 1), 0)
             ).astype(jnp.float32)
    scaled_cfg = cfg_ref[:, :, :] * cfgw_ref[0, :][None, None, :]
    cfg_flat = scaled_cfg.reshape(_G * _C, _NUM_CFG_FEAT)
    cfg_all = (jax.lax.dot_general(
                   cfg_flat, wcfg_ref[:, :], (((1,), (1,)), ((), ())),
                   preferred_element_type=jnp.float32)
               + bcfg_ref[0, :][None, :]).reshape(_G, _C, _DIM)
    cfg_pn_flat = jax.lax.dot_general(
        ohT_g, cfg_all.reshape(_G, _C * _DIM), (((0,), (0,)), ((), ())),
        preferred_element_type=jnp.float32)
    res_flat = cfg_pn_flat + jnp.concatenate([x] * _C, axis=1)
    out_ref[:, :, :] = res_flat.reshape(_BLOCK, _C, _DIM)


def kernel(op_code, op_feats, config_feats, batch_idx, op_weights,
           config_weights, emb_table, W_op, b_op, W_cfg, b_cfg):
    n = op_feats.shape[0]
    nb = -(-n // _BLOCK)
    pad = nb * _BLOCK - n
    code_p = jnp.pad(op_code.astype(jnp.int32), (0, pad))
    bidx_p = jnp.pad(batch_idx.astype(jnp.int32), (0, pad))
    code3 = code_p.reshape(nb, 1, _BLOCK)
    bidx3 = bidx_p.reshape(nb, 1, _BLOCK)
    cfgw2 = config_weights.reshape(1, _NUM_CFG_FEAT)
    bop2 = b_op.reshape(1, _DIM)
    bcfg2 = b_cfg.reshape(1, _DIM)

    grid = (nb,)

    def whole(shape):
        return pl.BlockSpec(shape, lambda i: (0,) * len(shape))

    out = pl.pallas_call(
        _fused_kernel,
        grid=grid,
        in_specs=[
            pl.BlockSpec((1, 1, _BLOCK), lambda i: (i, 0, 0)),
            pl.BlockSpec((1, 1, _BLOCK), lambda i: (i, 0, 0)),
            pl.BlockSpec((_BLOCK, _NUM_FEAT), lambda i: (i, 0)),
            whole((_G, _C, _NUM_CFG_FEAT)),
            whole((1, 1)),
            whole((1, _NUM_CFG_FEAT)),
            whole((_NUM_OPS, _DIM)),
            whole((_DIM, _NUM_FEAT)),
            whole((1, _DIM)),
            whole((_DIM, _NUM_CFG_FEAT)),
            whole((1, _DIM)),
        ],
        out_specs=pl.BlockSpec((_BLOCK, _C, _DIM), lambda i: (i, 0, 0)),
        out_shape=jax.ShapeDtypeStruct((n, _C, _DIM), jnp.float32),
    )(code3, bidx3, op_feats, config_feats, op_weights, cfgw2,
      emb_table, W_op, bop2, W_cfg, bcfg2)
    return out


# speedup vs baseline: 1.2324x; 1.0097x over previous
import jax
import jax.numpy as jnp
from jax.experimental import pallas as pl

_G = 16
_C = 8
_NUM_FEAT = 123
_NUM_CFG_FEAT = 18
_NUM_OPS = 125
_DIM = 128

_BLOCK = 5000


def _fused_kernel(code_ref, bidx_ref, opf_ref, cfg_ref, opw_ref, cfgw_ref,
                  emb_ref, wop_ref, bop_ref, wcfg_ref, bcfg_ref, out_ref):
    code = code_ref[0, :, :]
    ohT_op = (code == jax.lax.broadcasted_iota(jnp.int32, (_NUM_OPS, 1), 0)
              ).astype(jnp.float32)
    row = jax.lax.dot_general(
        ohT_op, emb_ref[:, :], (((0,), (0,)), ((), ())),
        preferred_element_type=jnp.float32)
    sq = jnp.sum(row * row, axis=1, keepdims=True)
    scale = jnp.where(sq > 1.0, jax.lax.rsqrt(sq), 1.0)
    op_emb = opw_ref[0, 0] * (row * scale)

    x = (jax.lax.dot_general(
            opf_ref[:, :], wop_ref[:, :], (((1,), (1,)), ((), ())),
            preferred_element_type=jnp.float32)
         + bop_ref[0, :][None, :] + op_emb)

    ohT_g = (bidx_ref[0, :, :] ==
             jax.lax.broadcasted_iota(jnp.int32, (_G, 1), 0)
             ).astype(jnp.float32)
    scaled_cfg = cfg_ref[:, :, :] * cfgw_ref[0, :][None, None, :]
    cfg_flat = scaled_cfg.reshape(_G * _C, _NUM_CFG_FEAT)
    cfg_all = (jax.lax.dot_general(
                   cfg_flat, wcfg_ref[:, :], (((1,), (1,)), ((), ())),
                   preferred_element_type=jnp.float32)
               + bcfg_ref[0, :][None, :]).reshape(_G, _C, _DIM)
    cfg_pn_flat = jax.lax.dot_general(
        ohT_g, cfg_all.reshape(_G, _C * _DIM), (((0,), (0,)), ((), ())),
        preferred_element_type=jnp.float32)
    res_flat = cfg_pn_flat + jnp.concatenate([x] * _C, axis=1)
    out_ref[:, :, :] = res_flat.reshape(_BLOCK, _C, _DIM)


def kernel(op_code, op_feats, config_feats, batch_idx, op_weights,
           config_weights, emb_table, W_op, b_op, W_cfg, b_cfg):
    n = op_feats.shape[0]
    nb = -(-n // _BLOCK)
    pad = nb * _BLOCK - n
    code_p = jnp.pad(op_code.astype(jnp.int32), (0, pad))
    bidx_p = jnp.pad(batch_idx.astype(jnp.int32), (0, pad))
    code3 = code_p.reshape(nb, 1, _BLOCK)
    bidx3 = bidx_p.reshape(nb, 1, _BLOCK)
    cfgw2 = config_weights.reshape(1, _NUM_CFG_FEAT)
    bop2 = b_op.reshape(1, _DIM)
    bcfg2 = b_cfg.reshape(1, _DIM)

    grid = (nb,)

    def whole(shape):
        return pl.BlockSpec(shape, lambda i: (0,) * len(shape))

    out = pl.pallas_call(
        _fused_kernel,
        grid=grid,
        in_specs=[
            pl.BlockSpec((1, 1, _BLOCK), lambda i: (i, 0, 0)),
            pl.BlockSpec((1, 1, _BLOCK), lambda i: (i, 0, 0)),
            pl.BlockSpec((_BLOCK, _NUM_FEAT), lambda i: (i, 0)),
            whole((_G, _C, _NUM_CFG_FEAT)),
            whole((1, 1)),
            whole((1, _NUM_CFG_FEAT)),
            whole((_NUM_OPS, _DIM)),
            whole((_DIM, _NUM_FEAT)),
            whole((1, _DIM)),
            whole((_DIM, _NUM_CFG_FEAT)),
            whole((1, _DIM)),
        ],
        out_specs=pl.BlockSpec((_BLOCK, _C, _DIM), lambda i: (i, 0, 0)),
        out_shape=jax.ShapeDtypeStruct((n, _C, _DIM), jnp.float32),
    )(code3, bidx3, op_feats, config_feats, op_weights, cfgw2,
      emb_table, W_op, bop2, W_cfg, bcfg2)
    return out
